# baseline (speedup 1.0000x reference)
import functools

import jax
import jax.numpy as jnp
from jax import lax
from jax.experimental import pallas as pl
from jax.experimental.pallas import tpu as pltpu
from jax.experimental.pallas import tpu_sc as plsc

B = 16384
ED = 10
NC, NS = 2, 16
NW = NC * NS
BPW = B // NW
CHUNK = 128
NCH = BPW // CHUNK


def _gather5(uid2, tid2, tca2, sid2, sca2, ue, tie, tce, sie, sce):
    mesh = plsc.VectorSubcoreMesh(core_axis_name="c", subcore_axis_name="s")
    out_t = [jax.ShapeDtypeStruct((B, ED), jnp.float32)] * 5
    scratch = ([pltpu.VMEM((NCH, CHUNK), jnp.int32) for _ in range(5)]
               + [pltpu.VMEM((BPW, ED), jnp.float32) for _ in range(5)]
               + [pltpu.SemaphoreType.DMA])

    @functools.partial(pl.kernel, out_type=out_t, mesh=mesh,
                       scratch_types=scratch,
                       compiler_params=pltpu.CompilerParams(
                           use_tc_tiling_on_sc=False))
    def k(uid_h, tid_h, tca_h, sid_h, sca_h,
          ue_h, tie_h, tce_h, sie_h, sce_h,
          o0, o1, o2, o3, o4,
          i0, i1, i2, i3, i4, r0, r1, r2, r3, r4, sem):
        wid = lax.axis_index("s") * NC + lax.axis_index("c")
        idx_hs = (uid_h, tid_h, tca_h, sid_h, sca_h)
        idx_vs = (i0, i1, i2, i3, i4)
        row_vs = (r0, r1, r2, r3, r4)
        tabs = (ue_h, tie_h, tce_h, sie_h, sce_h)
        outs = (o0, o1, o2, o3, o4)
        for t in range(5):
            pltpu.sync_copy(idx_hs[t].at[pl.ds(wid * NCH, NCH)], idx_vs[t])
        cps = []
        for t in range(5):
            for j in range(NCH):
                cps.append(pltpu.async_copy(
                    tabs[t].at[idx_vs[t].at[j]],
                    row_vs[t].at[pl.ds(j * CHUNK, CHUNK)], sem))
        for cp in cps:
            cp.wait()
        for t in range(5):
            pltpu.sync_copy(row_vs[t], outs[t].at[pl.ds(wid * BPW, BPW)])

    return k(uid2, tid2, tca2, sid2, sca2, ue, tie, tce, sie, sce)


def _mlp_body(eu, ti, tc, si, sc,
              su, wsm, wsh, tu, wtm, wth, hm, hh,
              ws1, h1, wt1, ws2, h2, wt2, ws3, h3, wt3,
              sw, sb, tw, tb, rs, rt):
    dT = lambda w, x: lax.dot_general(w[...], x, (((1,), (1,)), ((), ())),
                                      preferred_element_type=jnp.float32)
    d = lambda w, x: lax.dot_general(w[...], x, (((1,), (0,)), ((), ())),
                                     preferred_element_type=jnp.float32)
    eu_, ti_, tc_, si_, sc_ = eu[...], ti[...], tc[...], si[...], sc[...]
    a_s = dT(su, eu_) + dT(wsm, si_) + dT(wsh, sc_) + dT(hm, ti_) + dT(hh, tc_)
    a_t = dT(tu, eu_) + dT(wtm, ti_) + dT(wth, tc_) + dT(hm, si_) + dT(hh, sc_)
    xs = jnp.maximum(a_s, 0.0)
    xt = jnp.maximum(a_t, 0.0)
    for (w, h, wt) in ((ws1, h1, wt1), (ws2, h2, wt2), (ws3, h3, wt3)):
        ns = jnp.maximum(d(w, xs) + d(h, xt), 0.0)
        nt = jnp.maximum(d(wt, xt) + d(h, xs), 0.0)
        xs, xt = ns, nt
    ls = d(sw, xs) + sb[...]
    lt = d(tw, xt) + tb[...]
    rs[...] = 1.0 / (1.0 + jnp.exp(-ls))
    rt[...] = 1.0 / (1.0 + jnp.exp(-lt))


def _mlp(eu, ti, tc, si, sc, mats, sw, sb, tw, tb, interpret=False):
    BB = 2048
    grid = (B // BB,)
    dspec = pl.BlockSpec((BB, ED), lambda i: (i, 0))
    wspec = lambda a: pl.BlockSpec(a.shape, lambda i: (0, 0))
    in_specs = ([dspec] * 5 + [wspec(m) for m in mats]
                + [wspec(sw), wspec(sb), wspec(tw), wspec(tb)])
    out_specs = [pl.BlockSpec((1, BB), lambda i: (0, i))] * 2
    out_shape = [jax.ShapeDtypeStruct((1, B), jnp.float32)] * 2
    return pl.pallas_call(
        _mlp_body, grid=grid, in_specs=in_specs, out_specs=out_specs,
        out_shape=out_shape, interpret=interpret,
    )(eu, ti, tc, si, sc, *mats, sw, sb, tw, tb)


def kernel(userid, t_can_id, t_can_cate, s_can_id, s_can_cate,
           user_emb, t_itemid_emb, t_itemcate_emb, s_itemid_emb, s_itemcate_emb,
           ws0, h0, wt0, ws1, h1, wt1, ws2, h2, wt2, ws3, h3, wt3,
           s_pred_w, s_pred_b, t_pred_w, t_pred_b):
    uid2 = userid.reshape(B // CHUNK, CHUNK)
    tid2 = t_can_id.reshape(B // CHUNK, CHUNK)
    tca2 = t_can_cate.reshape(B // CHUNK, CHUNK)
    sid2 = s_can_id.reshape(B // CHUNK, CHUNK)
    sca2 = s_can_cate.reshape(B // CHUNK, CHUNK)
    eu, ti, tc, si, sc = _gather5(uid2, tid2, tca2, sid2, sca2,
                                  user_emb, t_itemid_emb, t_itemcate_emb,
                                  s_itemid_emb, s_itemcate_emb)
    mats = (ws0[:, :ED] + h0[:, :ED],
            ws0[:, ED:2 * ED], ws0[:, 2 * ED:],
            wt0[:, :ED] + h0[:, :ED],
            wt0[:, ED:2 * ED], wt0[:, 2 * ED:],
            h0[:, ED:2 * ED], h0[:, 2 * ED:],
            ws1, h1, wt1, ws2, h2, wt2, ws3, h3, wt3)
    rs, rt = _mlp(eu, ti, tc, si, sc, mats,
                  s_pred_w, s_pred_b.reshape(1, 1),
                  t_pred_w, t_pred_b.reshape(1, 1))
    return rs.reshape(B), rt.reshape(B)
